# baseline (speedup 1.0000x reference)
import functools

import jax
import jax.numpy as jnp
from jax import lax
from jax.experimental import pallas as pl
from jax.experimental.pallas import tpu as pltpu
from jax.experimental.pallas import tpu_sc as plsc

_B = 65536
_J_IN = 144
_J_OUT = 118
_NC, _NS = 2, 16
_NW = _NC * _NS
_CHUNK = 4096
_NCHUNK = _B // _CHUNK
_NTASK = 3 * _J_OUT * _NCHUNK
_STEPS = -(-_NTASK // _NW)
_NBUF = 12
_DEPTH = 6

_mesh = plsc.VectorSubcoreMesh(core_axis_name="c", subcore_axis_name="s")


@functools.partial(
    pl.kernel,
    out_type=jax.ShapeDtypeStruct((3, _J_OUT, _B), jnp.float32),
    mesh=_mesh,
    scratch_types=[
        pltpu.VMEM((144,), jnp.int32),
        *[pltpu.VMEM((_CHUNK,), jnp.float32) for _ in range(_NBUF)],
        *[pltpu.SemaphoreType.DMA for _ in range(2 * _NBUF)],
    ],
    compiler_params=pltpu.CompilerParams(use_tc_tiling_on_sc=True,
                                         needs_layout_passes=False),
)
def _sc_copy(x_hbm, jm_hbm, out_hbm, jm_v, *bufs_and_sems):
    bufs = bufs_and_sems[:_NBUF]
    isems = bufs_and_sems[_NBUF:2 * _NBUF]
    osems = bufs_and_sems[2 * _NBUF:3 * _NBUF]
    wid = lax.axis_index("s") * _NC + lax.axis_index("c")
    pltpu.sync_copy(jm_hbm, jm_v)

    def addr(i):
        t = wid + i * _NW
        k = t % _NCHUNK
        p = t // _NCHUNK
        c = p // _J_OUT
        r = p % _J_OUT
        j = jm_v[pl.ds(r, 16)][0]
        return c, r, j, k * _CHUNK

    def in_copy(i):
        c, _, j, off = addr(i)
        return pltpu.make_async_copy(
            x_hbm.at[c, j, pl.ds(off, _CHUNK)], bufs[i % _NBUF],
            isems[i % _NBUF])

    def out_copy(i):
        c, r, _, off = addr(i)
        return pltpu.make_async_copy(
            bufs[i % _NBUF], out_hbm.at[c, r, pl.ds(off, _CHUNK)],
            osems[i % _NBUF])

    def step(i):
        in_copy(i).wait()
        out_copy(i).start()
        nxt = i + _DEPTH
        if nxt < _STEPS:
            if nxt - _NBUF >= 0:
                out_copy(nxt - _NBUF).wait()

            def start_next():
                in_copy(nxt).start()
            if (nxt + 1) * _NW <= _NTASK:
                start_next()
            else:
                pl.when(wid < _NTASK - nxt * _NW)(start_next)

    for i in range(_DEPTH):
        if (i + 1) * _NW <= _NTASK:
            in_copy(i).start()
        else:
            pl.when(wid < _NTASK - i * _NW)(lambda i=i: in_copy(i).start())

    for i in range(_STEPS):
        if (i + 1) * _NW <= _NTASK:
            step(i)
        else:
            pl.when(wid < _NTASK - i * _NW)(lambda i=i: step(i))

    for i in range(max(_STEPS - _NBUF, 0), _STEPS):
        if (i + 1) * _NW <= _NTASK:
            out_copy(i).wait()
        else:
            pl.when(wid < _NTASK - i * _NW)(lambda i=i: out_copy(i).wait())


@jax.jit
def kernel(joints, joint_maps):
    xt = joints.transpose(2, 1, 0)
    jm = jnp.pad(joint_maps.astype(jnp.int32), (0, 144 - _J_OUT))
    out_t = _sc_copy(xt, jm)
    return out_t.transpose(2, 1, 0)


# speedup vs baseline: 1.0773x; 1.0773x over previous
import functools

import jax
import jax.numpy as jnp
from jax import lax
from jax.experimental import pallas as pl
from jax.experimental.pallas import tpu as pltpu
from jax.experimental.pallas import tpu_sc as plsc

_B = 65536
_J_IN = 144
_J_OUT = 118
_NC, _NS = 2, 16
_NW = _NC * _NS
_CHUNK = 16384
_NCHUNK = _B // _CHUNK
_NTASK = 3 * _J_OUT * _NCHUNK
_STEPS = -(-_NTASK // _NW)
_NBUF = 6
_DEPTH = 4

_mesh = plsc.VectorSubcoreMesh(core_axis_name="c", subcore_axis_name="s")


@functools.partial(
    pl.kernel,
    out_type=jax.ShapeDtypeStruct((3, _J_OUT, _B), jnp.float32),
    mesh=_mesh,
    scratch_types=[
        pltpu.VMEM((144,), jnp.int32),
        *[pltpu.VMEM((_CHUNK,), jnp.float32) for _ in range(_NBUF)],
        *[pltpu.SemaphoreType.DMA for _ in range(2 * _NBUF)],
    ],
    compiler_params=pltpu.CompilerParams(use_tc_tiling_on_sc=True,
                                         needs_layout_passes=False),
)
def _sc_copy(x_hbm, jm_hbm, out_hbm, jm_v, *bufs_and_sems):
    bufs = bufs_and_sems[:_NBUF]
    isems = bufs_and_sems[_NBUF:2 * _NBUF]
    osems = bufs_and_sems[2 * _NBUF:3 * _NBUF]
    wid = lax.axis_index("s") * _NC + lax.axis_index("c")
    pltpu.sync_copy(jm_hbm, jm_v)

    def addr(i):
        t = wid + i * _NW
        k = t % _NCHUNK
        p = t // _NCHUNK
        c = p // _J_OUT
        r = p % _J_OUT
        j = jm_v[pl.ds(r, 16)][0]
        return c, r, j, k * _CHUNK

    def in_copy(i):
        c, _, j, off = addr(i)
        return pltpu.make_async_copy(
            x_hbm.at[c, j, pl.ds(off, _CHUNK)], bufs[i % _NBUF],
            isems[i % _NBUF])

    def out_copy(i):
        c, r, _, off = addr(i)
        return pltpu.make_async_copy(
            bufs[i % _NBUF], out_hbm.at[c, r, pl.ds(off, _CHUNK)],
            osems[i % _NBUF])

    def step(i):
        in_copy(i).wait()
        out_copy(i).start()
        nxt = i + _DEPTH
        if nxt < _STEPS:
            if nxt - _NBUF >= 0:
                out_copy(nxt - _NBUF).wait()

            def start_next():
                in_copy(nxt).start()
            if (nxt + 1) * _NW <= _NTASK:
                start_next()
            else:
                pl.when(wid < _NTASK - nxt * _NW)(start_next)

    for i in range(_DEPTH):
        if (i + 1) * _NW <= _NTASK:
            in_copy(i).start()
        else:
            pl.when(wid < _NTASK - i * _NW)(lambda i=i: in_copy(i).start())

    for i in range(_STEPS):
        if (i + 1) * _NW <= _NTASK:
            step(i)
        else:
            pl.when(wid < _NTASK - i * _NW)(lambda i=i: step(i))

    for i in range(max(_STEPS - _NBUF, 0), _STEPS):
        if (i + 1) * _NW <= _NTASK:
            out_copy(i).wait()
        else:
            pl.when(wid < _NTASK - i * _NW)(lambda i=i: out_copy(i).wait())


@jax.jit
def kernel(joints, joint_maps):
    xt = joints.transpose(2, 1, 0)
    jm = jnp.pad(joint_maps.astype(jnp.int32), (0, 144 - _J_OUT))
    out_t = _sc_copy(xt, jm)
    return out_t.transpose(2, 1, 0)
